# baseline (speedup 1.0000x reference)
import jax
import jax.numpy as jnp
from jax import lax
from jax.experimental import pallas as pl
from jax.experimental.pallas import tpu as pltpu

_B = 8
_IN = 128
_H = 800
_C = 800
_A = 128
_HEADS = 8
_DH = _C // _HEADS
_RADIUS2 = 64.0
_TOTAL = 256
_L = _TOTAL

_PREC = lax.Precision.DEFAULT
_JB = 16


def _dot(a, b):
    return jnp.dot(a, b, precision=_PREC, preferred_element_type=jnp.float32)


def _dot_t(a, b):
    return lax.dot_general(a, b, (((1,), (1,)), ((), ())),
                           precision=_PREC, preferred_element_type=jnp.float32)


def _rnn_kernel(s_ref, inputs_ref, cnt_ref,
                wih_r_ref, wih_z_ref, wih_n_ref, bih_ref,
                whh_r_ref, whh_z_ref, whh_n_ref, bhh_ref,
                wa_ref, ba_ref, w1_ref, b1_ref,
                w2_ref, mw_ref, wv_ref,
                outp_ref, outa_ref, poss_ref, v_ref, h1_ref):
    f32 = jnp.float32
    poss_ref[...] = jnp.zeros((_L, _B, 9), f32)
    v_ref[...] = jnp.zeros((_L, _B, _C), f32)
    h1_ref[...] = jnp.zeros((_L, _B, _A), f32)

    e_mat = (lax.broadcasted_iota(jnp.int32, (_HEADS, _C), 1) // _DH ==
             lax.broadcasted_iota(jnp.int32, (_HEADS, _C), 0)).astype(f32)
    p_mat = (lax.broadcasted_iota(jnp.int32, (_B, _JB * _B), 1) % _B ==
             lax.broadcasted_iota(jnp.int32, (_B, _JB * _B), 0)).astype(f32)
    jidx = lax.broadcasted_iota(jnp.int32, (_JB, _B, 1), 0)

    bih = bih_ref[...]
    bhh = bhh_ref[...]
    w2 = w2_ref[...]
    mw = mw_ref[...]
    b1 = b1_ref[...]
    ba = ba_ref[...]
    w1n = w1_ref[931:934, :]
    wvn = wv_ref[931:934, :]

    def _in_rows(t):
        return jnp.concatenate(
            [inputs_ref[pl.ds(jnp.minimum(s_ref[1 + b] + t, _TOTAL - 1), 1)]
             for b in range(_B)], axis=0)

    def body(t, carry):
        h, ctx, dt_raw = carry
        tm1 = jnp.maximum(t - 1, 0)
        prev_row = poss_ref[tm1]
        is0 = (t == 0)
        base = jnp.where(is0, jnp.zeros((_B, 3), f32), prev_row[:, 6:9])

        def blk(i, acc):
            num_a, den_a = acc
            possv = poss_ref[pl.ds(i * _JB, _JB)]
            dsq = possv - prev_row[None, :, :]
            dsq = dsq * dsq
            s3 = dsq[:, :, 0:3] + dsq[:, :, 3:6] + dsq[:, :, 6:9]
            ndist = jnp.sqrt(s3).reshape(_JB * _B, 3)
            d2 = jnp.sum(dsq[:, :, 3:6], axis=2, keepdims=True)
            maskf = ((i * _JB + jidx < t) &
                     (d2 <= _RADIUS2)).astype(f32).reshape(_JB * _B, 1)

            h1v = h1_ref[pl.ds(i * _JB, _JB)].reshape(_JB * _B, _A)
            hid = jnp.tanh(h1v + _dot(ndist, w1n))
            logits = _dot(hid, w2) - mw
            w = jnp.exp(logits) * maskf
            den_a = den_a + _dot(p_mat, w)
            wfull = _dot(w, e_mat)
            vals = (v_ref[pl.ds(i * _JB, _JB)].reshape(_JB * _B, _C) +
                    _dot(ndist, wvn))
            num_a = num_a + _dot(p_mat, wfull * vals)
            return num_a, den_a

        num0, den0 = blk(0, (jnp.zeros((_B, _C), f32),
                             jnp.zeros((_B, _HEADS), f32)))
        nblk = (t + _JB - 1) // _JB
        num, den = lax.fori_loop(1, nblk, blk, (num0, den0))
        den_full = _dot(den, e_mat)
        ctx_new = num / jnp.maximum(den_full, 1e-30)
        ctx_new = jnp.where(is0, jnp.zeros((_B, _C), f32), ctx_new)

        dt = jnp.where(t < cnt_ref[...], dt_raw, 0.0)
        x = jnp.concatenate([dt, ctx], axis=1)

        gi_r = _dot(x, wih_r_ref[...]) + bih[:, 0:_H]
        gi_z = _dot(x, wih_z_ref[...]) + bih[:, _H:2 * _H]
        gi_n = _dot(x, wih_n_ref[...]) + bih[:, 2 * _H:3 * _H]
        gh_r = _dot(h, whh_r_ref[...]) + bhh[:, 0:_H]
        gh_z = _dot(h, whh_z_ref[...]) + bhh[:, _H:2 * _H]
        gh_n = _dot(h, whh_n_ref[...]) + bhh[:, 2 * _H:3 * _H]
        r = jax.nn.sigmoid(gi_r + gh_r)
        z = jax.nn.sigmoid(gi_z + gh_z)
        n = jnp.tanh(gi_n + r * gh_n)
        h_new = (1.0 - z) * n + z * h

        angle = jnp.pi * jnp.tanh(_dot(h_new, wa_ref[...]) + ba)

        cosv = jnp.cos(angle)
        sinv = jnp.sin(angle)
        rn = lax.rsqrt(cosv * cosv + sinv * sinv + 0.01)
        ux = 1.5 * cosv * rn
        uy = 1.5 * sinv * rn
        uz = 0.15 * rn
        a0x = base[:, 0:1] + ux[:, 0:1]
        a0y = base[:, 1:2] + uy[:, 0:1]
        a0z = base[:, 2:3] + uz[:, 0:1]
        a1x = a0x + ux[:, 1:2]
        a1y = a0y + uy[:, 1:2]
        a1z = a0z + uz[:, 1:2]
        a2x = a1x + ux[:, 2:3]
        a2y = a1y + uy[:, 2:3]
        a2z = a1z + uz[:, 2:3]
        prow = jnp.concatenate(
            [a0x, a0y, a0z, a1x, a1y, a1z, a2x, a2y, a2z], axis=1)

        poss_ref[t] = prow
        for b in range(_B):
            @pl.when(t < s_ref[9 + b])
            def _(b=b):
                o = s_ref[1 + b] + t
                outp_ref[pl.ds(o, 1), :] = prow[b:b + 1, :]
                outa_ref[pl.ds(o, 1), :] = angle[b:b + 1, :]
        frow = jnp.concatenate(
            [h_new, dt, angle, jnp.zeros((_B, 3), f32)], axis=1)
        v_ref[t] = _dot(frow, wv_ref[...])
        h1_ref[t] = _dot(frow, w1_ref[...]) + b1
        return h_new, ctx_new, _in_rows(t + 1)

    h0 = jnp.zeros((_B, _H), f32)
    c0 = jnp.zeros((_B, _C), f32)
    lax.fori_loop(0, s_ref[0], body, (h0, c0, _in_rows(0)))


def kernel(inputs, structure, W_ih, b_ih, W_hh, b_hh, Wa, ba, W1, b1, W2, Wv):
    f32 = jnp.float32
    counts = jnp.sum(
        (structure[None, :] == jnp.arange(_B, dtype=structure.dtype)[:, None]),
        axis=1).astype(jnp.int32)
    starts = jnp.concatenate(
        [jnp.zeros((1,), counts.dtype), jnp.cumsum(counts)[:-1]])
    sarr = jnp.concatenate(
        [jnp.max(counts).reshape(1), starts, counts])

    WihT = W_ih.T.astype(f32)
    WhhT = W_hh.T.astype(f32)
    mw = jnp.sum(jnp.abs(W2), axis=0).reshape(1, _HEADS)

    bf16 = jnp.bfloat16
    outp, outa = pl.pallas_call(
        _rnn_kernel,
        in_specs=[pl.BlockSpec(memory_space=pltpu.SMEM)] +
                 [pl.BlockSpec(memory_space=pltpu.VMEM)] * 17,
        out_specs=[pl.BlockSpec(memory_space=pltpu.VMEM)] * 2,
        out_shape=[
            jax.ShapeDtypeStruct((_TOTAL, 9), f32),
            jax.ShapeDtypeStruct((_TOTAL, 3), f32),
        ],
        scratch_shapes=[
            pltpu.VMEM((_L, _B, 9), f32),
            pltpu.VMEM((_L, _B, _C), f32),
            pltpu.VMEM((_L, _B, _A), f32),
        ],
    )(
        sarr, inputs.astype(f32), counts.reshape(_B, 1),
        WihT[:, 0:_H].astype(bf16), WihT[:, _H:2 * _H].astype(bf16),
        WihT[:, 2 * _H:3 * _H].astype(bf16),
        b_ih.reshape(1, 3 * _H),
        WhhT[:, 0:_H].astype(bf16), WhhT[:, _H:2 * _H].astype(bf16),
        WhhT[:, 2 * _H:3 * _H].astype(bf16),
        b_hh.reshape(1, 3 * _H),
        Wa, ba.reshape(1, 3),
        W1.astype(bf16), b1.reshape(1, _A),
        W2, mw, Wv.astype(bf16),
    )
    return outp.reshape(_TOTAL, 3, 3), outa


# speedup vs baseline: 1.0539x; 1.0539x over previous
import jax
import jax.numpy as jnp
from jax import lax
from jax.experimental import pallas as pl
from jax.experimental.pallas import tpu as pltpu

_B = 8
_IN = 128
_H = 800
_C = 800
_A = 128
_HEADS = 8
_DH = _C // _HEADS
_RADIUS2 = 64.0
_TOTAL = 256
_L = _TOTAL

_PREC = lax.Precision.DEFAULT
_JB = 32


def _dot(a, b):
    return jnp.dot(a, b, precision=_PREC, preferred_element_type=jnp.float32)


def _dot_t(a, b):
    return lax.dot_general(a, b, (((1,), (1,)), ((), ())),
                           precision=_PREC, preferred_element_type=jnp.float32)


def _rnn_kernel(s_ref, inputs_ref, cnt_ref,
                wih_r_ref, wih_z_ref, wih_n_ref, bih_ref,
                whh_r_ref, whh_z_ref, whh_n_ref, bhh_ref,
                wa_ref, ba_ref, w1_ref, b1_ref,
                w2_ref, mw_ref, wv_ref,
                outp_ref, outa_ref, poss_ref, v_ref, h1_ref):
    f32 = jnp.float32
    poss_ref[...] = jnp.zeros((_L, _B, 9), f32)
    v_ref[...] = jnp.zeros((_L, _B, _C), f32)
    h1_ref[...] = jnp.zeros((_L, _B, _A), f32)

    e_mat = (lax.broadcasted_iota(jnp.int32, (_HEADS, _C), 1) // _DH ==
             lax.broadcasted_iota(jnp.int32, (_HEADS, _C), 0)).astype(f32)
    p_mat = (lax.broadcasted_iota(jnp.int32, (_B, _JB * _B), 1) % _B ==
             lax.broadcasted_iota(jnp.int32, (_B, _JB * _B), 0)).astype(f32)
    jidx = lax.broadcasted_iota(jnp.int32, (_JB, _B, 1), 0)

    bih = bih_ref[...]
    bhh = bhh_ref[...]
    w2 = w2_ref[...]
    mw = mw_ref[...]
    b1 = b1_ref[...]
    ba = ba_ref[...]
    w1n = w1_ref[931:934, :]
    wvn = wv_ref[931:934, :]

    def _in_rows(t):
        return jnp.concatenate(
            [inputs_ref[pl.ds(jnp.minimum(s_ref[1 + b] + t, _TOTAL - 1), 1)]
             for b in range(_B)], axis=0)

    def body(t, carry):
        h, ctx, dt_raw = carry
        tm1 = jnp.maximum(t - 1, 0)
        prev_row = poss_ref[tm1]
        is0 = (t == 0)
        base = jnp.where(is0, jnp.zeros((_B, 3), f32), prev_row[:, 6:9])

        def blk(i, acc):
            num_a, den_a = acc
            possv = poss_ref[pl.ds(i * _JB, _JB)]
            dsq = possv - prev_row[None, :, :]
            dsq = dsq * dsq
            s3 = dsq[:, :, 0:3] + dsq[:, :, 3:6] + dsq[:, :, 6:9]
            ndist = jnp.sqrt(s3).reshape(_JB * _B, 3)
            d2 = jnp.sum(dsq[:, :, 3:6], axis=2, keepdims=True)
            maskf = ((i * _JB + jidx < t) &
                     (d2 <= _RADIUS2)).astype(f32).reshape(_JB * _B, 1)

            h1v = h1_ref[pl.ds(i * _JB, _JB)].reshape(_JB * _B, _A)
            hid = jnp.tanh(h1v + _dot(ndist, w1n))
            logits = _dot(hid, w2) - mw
            w = jnp.exp(logits) * maskf
            den_a = den_a + _dot(p_mat, w)
            wfull = _dot(w, e_mat)
            vals = (v_ref[pl.ds(i * _JB, _JB)].reshape(_JB * _B, _C) +
                    _dot(ndist, wvn))
            num_a = num_a + _dot(p_mat, wfull * vals)
            return num_a, den_a

        num0, den0 = blk(0, (jnp.zeros((_B, _C), f32),
                             jnp.zeros((_B, _HEADS), f32)))
        nblk = (t + _JB - 1) // _JB
        num, den = lax.fori_loop(1, nblk, blk, (num0, den0))
        den_full = _dot(den, e_mat)
        ctx_new = num / jnp.maximum(den_full, 1e-30)
        ctx_new = jnp.where(is0, jnp.zeros((_B, _C), f32), ctx_new)

        dt = jnp.where(t < cnt_ref[...], dt_raw, 0.0)
        x = jnp.concatenate([dt, ctx], axis=1)

        gi_r = _dot(x, wih_r_ref[...]) + bih[:, 0:_H]
        gi_z = _dot(x, wih_z_ref[...]) + bih[:, _H:2 * _H]
        gi_n = _dot(x, wih_n_ref[...]) + bih[:, 2 * _H:3 * _H]
        gh_r = _dot(h, whh_r_ref[...]) + bhh[:, 0:_H]
        gh_z = _dot(h, whh_z_ref[...]) + bhh[:, _H:2 * _H]
        gh_n = _dot(h, whh_n_ref[...]) + bhh[:, 2 * _H:3 * _H]
        r = jax.nn.sigmoid(gi_r + gh_r)
        z = jax.nn.sigmoid(gi_z + gh_z)
        n = jnp.tanh(gi_n + r * gh_n)
        h_new = (1.0 - z) * n + z * h

        angle = jnp.pi * jnp.tanh(_dot(h_new, wa_ref[...]) + ba)

        cosv = jnp.cos(angle)
        sinv = jnp.sin(angle)
        rn = lax.rsqrt(cosv * cosv + sinv * sinv + 0.01)
        ux = 1.5 * cosv * rn
        uy = 1.5 * sinv * rn
        uz = 0.15 * rn
        a0x = base[:, 0:1] + ux[:, 0:1]
        a0y = base[:, 1:2] + uy[:, 0:1]
        a0z = base[:, 2:3] + uz[:, 0:1]
        a1x = a0x + ux[:, 1:2]
        a1y = a0y + uy[:, 1:2]
        a1z = a0z + uz[:, 1:2]
        a2x = a1x + ux[:, 2:3]
        a2y = a1y + uy[:, 2:3]
        a2z = a1z + uz[:, 2:3]
        prow = jnp.concatenate(
            [a0x, a0y, a0z, a1x, a1y, a1z, a2x, a2y, a2z], axis=1)

        poss_ref[t] = prow
        for b in range(_B):
            @pl.when(t < s_ref[9 + b])
            def _(b=b):
                o = s_ref[1 + b] + t
                outp_ref[pl.ds(o, 1), :] = prow[b:b + 1, :]
                outa_ref[pl.ds(o, 1), :] = angle[b:b + 1, :]
        frow = jnp.concatenate(
            [h_new, dt, angle, jnp.zeros((_B, 3), f32)], axis=1)
        v_ref[t] = _dot(frow, wv_ref[...])
        h1_ref[t] = _dot(frow, w1_ref[...]) + b1
        return h_new, ctx_new, _in_rows(t + 1)

    h0 = jnp.zeros((_B, _H), f32)
    c0 = jnp.zeros((_B, _C), f32)
    def pair(i, carry):
        return body(2 * i + 1, body(2 * i, carry))
    lax.fori_loop(0, (s_ref[0] + 1) // 2, pair, (h0, c0, _in_rows(0)))


def kernel(inputs, structure, W_ih, b_ih, W_hh, b_hh, Wa, ba, W1, b1, W2, Wv):
    f32 = jnp.float32
    counts = jnp.sum(
        (structure[None, :] == jnp.arange(_B, dtype=structure.dtype)[:, None]),
        axis=1).astype(jnp.int32)
    starts = jnp.concatenate(
        [jnp.zeros((1,), counts.dtype), jnp.cumsum(counts)[:-1]])
    sarr = jnp.concatenate(
        [jnp.max(counts).reshape(1), starts, counts])

    WihT = W_ih.T.astype(f32)
    WhhT = W_hh.T.astype(f32)
    mw = jnp.sum(jnp.abs(W2), axis=0).reshape(1, _HEADS)

    bf16 = jnp.bfloat16
    outp, outa = pl.pallas_call(
        _rnn_kernel,
        in_specs=[pl.BlockSpec(memory_space=pltpu.SMEM)] +
                 [pl.BlockSpec(memory_space=pltpu.VMEM)] * 17,
        out_specs=[pl.BlockSpec(memory_space=pltpu.VMEM)] * 2,
        out_shape=[
            jax.ShapeDtypeStruct((_TOTAL, 9), f32),
            jax.ShapeDtypeStruct((_TOTAL, 3), f32),
        ],
        scratch_shapes=[
            pltpu.VMEM((_L, _B, 9), f32),
            pltpu.VMEM((_L, _B, _C), f32),
            pltpu.VMEM((_L, _B, _A), f32),
        ],
    )(
        sarr, inputs.astype(f32), counts.reshape(_B, 1),
        WihT[:, 0:_H].astype(bf16), WihT[:, _H:2 * _H].astype(bf16),
        WihT[:, 2 * _H:3 * _H].astype(bf16),
        b_ih.reshape(1, 3 * _H),
        WhhT[:, 0:_H].astype(bf16), WhhT[:, _H:2 * _H].astype(bf16),
        WhhT[:, 2 * _H:3 * _H].astype(bf16),
        b_hh.reshape(1, 3 * _H),
        Wa, ba.reshape(1, 3),
        W1.astype(bf16), b1.reshape(1, _A),
        W2, mw, Wv.astype(bf16),
    )
    return outp.reshape(_TOTAL, 3, 3), outa
